# Initial kernel scaffold; baseline (speedup 1.0000x reference)
import jax
import jax.numpy as jnp
from jax.experimental import pallas as pl


def kernel(feat, intend, last_nodes, segment_ids, bn_gamma, bn_beta, W_u, W_v, b_v, W_i, b_i, W_e):
    raise NotImplementedError("write your pallas kernel here")


# speedup vs baseline: 2.8455x; 2.8455x over previous
import functools

import jax
import jax.numpy as jnp
from jax.experimental import pallas as pl
from jax.experimental.pallas import tpu as pltpu

N = 100000
B = 1024
D = 128
H = 128

K1_BLK = 10000
K4_BLK = 2000
K1_STEPS = N // K1_BLK
K4_STEPS = N // K4_BLK


def _stats_kernel(feat_ref, out_ref):
    i = pl.program_id(0)

    @pl.when(i == 0)
    def _():
        out_ref[...] = jnp.zeros_like(out_ref)

    f = feat_ref[...]
    s = jnp.sum(f, axis=0)
    q = jnp.sum(f * f, axis=0)
    upd = jnp.concatenate(
        [s[None, :], q[None, :], jnp.zeros((6, D), dtype=jnp.float32)], axis=0)
    out_ref[...] += upd


def _gather_kernel(last_ref, feat_ref, out_ref):
    del last_ref
    out_ref[...] = feat_ref[...]


def _prep_kernel(stats_ref, gath_ref, intend_ref, wu_ref, wv_ref, wi_ref,
                 bv_ref, bi_ref, gamma_ref, beta_ref, we_ref,
                 cvec_ref, wu2_ref, params_ref):
    stats = stats_ref[...]
    mean = stats[0:1, :] / N
    ex2 = stats[1:2, :] / N
    var = ex2 - mean * mean
    g = gamma_ref[...] * jax.lax.rsqrt(var + 1e-5)
    boff = beta_ref[...] - mean * g

    wu = wu_ref[...]
    wu2 = wu * g
    bu2 = jax.lax.dot_general(boff, wu, (((1,), (1,)), ((), ())),
                              preferred_element_type=jnp.float32)

    wi2 = wi_ref[...] * g
    bi2 = bi_ref[...] + jax.lax.dot_general(boff, wi_ref[...],
                                            (((1,), (1,)), ((), ())),
                                            preferred_element_type=jnp.float32)

    cvec = (jax.lax.dot_general(gath_ref[...], wi2, (((1,), (1,)), ((), ())),
                                preferred_element_type=jnp.float32)
            + jax.lax.dot_general(intend_ref[...], wv_ref[...],
                                  (((1,), (1,)), ((), ())),
                                  preferred_element_type=jnp.float32)
            + bv_ref[...] + bi2)

    cvec_ref[...] = cvec
    wu2_ref[...] = wu2
    params_ref[...] = jnp.concatenate(
        [bu2, g, boff, we_ref[...],
         jnp.zeros((4, 128), dtype=jnp.float32)], axis=0)


def _main_kernel(feat_ref, ids_ref, cvec_ref, wu2_ref, params_ref,
                 rst_ref, den_ref):
    i = pl.program_id(0)

    @pl.when(i == 0)
    def _():
        rst_ref[...] = jnp.zeros_like(rst_ref)
        den_ref[...] = jnp.zeros_like(den_ref)

    f = feat_ref[...]
    u = jax.lax.dot_general(f, wu2_ref[...], (((1,), (1,)), ((), ())),
                            preferred_element_type=jnp.float32)
    u = u + params_ref[0:1, :]

    ids = ids_ref[0, 0, :]
    onehot = (ids[:, None] ==
              jax.lax.broadcasted_iota(jnp.int32, (K4_BLK, B), 1)
              ).astype(jnp.float32)
    cmat = jnp.dot(onehot, cvec_ref[...],
                   preferred_element_type=jnp.float32)

    s = jax.nn.sigmoid(u + cmat)
    e = jnp.sum(s * params_ref[3:4, :], axis=1, keepdims=True)
    w = jnp.exp(e)

    rst_ref[...] += jax.lax.dot_general(onehot, f * w,
                                        (((0,), (0,)), ((), ())),
                                        preferred_element_type=jnp.float32)
    den_ref[...] += jax.lax.dot_general(onehot, w,
                                        (((0,), (0,)), ((), ())),
                                        preferred_element_type=jnp.float32)

    @pl.when(i == K4_STEPS - 1)
    def _():
        den = den_ref[...]
        g = params_ref[1:2, :]
        boff = params_ref[2:3, :]
        safe = jnp.where(den > 0, den, 1.0)
        rst_ref[...] = jnp.where(den > 0,
                                 rst_ref[...] / safe * g + boff,
                                 0.0)


@jax.jit
def kernel(feat, intend, last_nodes, segment_ids, bn_gamma, bn_beta,
           W_u, W_v, b_v, W_i, b_i, W_e):
    feat = feat.astype(jnp.float32)

    stats = pl.pallas_call(
        _stats_kernel,
        grid=(K1_STEPS,),
        in_specs=[pl.BlockSpec((K1_BLK, D), lambda i: (i, 0))],
        out_specs=pl.BlockSpec((8, D), lambda i: (0, 0)),
        out_shape=jax.ShapeDtypeStruct((8, D), jnp.float32),
    )(feat)

    feat3 = feat.reshape(N, 1, D)
    gath3 = pl.pallas_call(
        _gather_kernel,
        grid_spec=pltpu.PrefetchScalarGridSpec(
            num_scalar_prefetch=1,
            grid=(B,),
            in_specs=[pl.BlockSpec((1, 1, D),
                                   lambda i, last_ref: (last_ref[i], 0, 0))],
            out_specs=pl.BlockSpec((1, 1, D),
                                   lambda i, last_ref: (i, 0, 0)),
        ),
        out_shape=jax.ShapeDtypeStruct((B, 1, D), jnp.float32),
    )(last_nodes.astype(jnp.int32), feat3)
    gath = gath3.reshape(B, D)

    cvec, wu2, params = pl.pallas_call(
        _prep_kernel,
        in_specs=[pl.BlockSpec((8, D), lambda: (0, 0)),
                  pl.BlockSpec((B, D), lambda: (0, 0)),
                  pl.BlockSpec((B, D), lambda: (0, 0)),
                  pl.BlockSpec((H, D), lambda: (0, 0)),
                  pl.BlockSpec((H, D), lambda: (0, 0)),
                  pl.BlockSpec((H, D), lambda: (0, 0)),
                  pl.BlockSpec((1, H), lambda: (0, 0)),
                  pl.BlockSpec((1, H), lambda: (0, 0)),
                  pl.BlockSpec((1, D), lambda: (0, 0)),
                  pl.BlockSpec((1, D), lambda: (0, 0)),
                  pl.BlockSpec((1, H), lambda: (0, 0))],
        out_specs=[pl.BlockSpec((B, H), lambda: (0, 0)),
                   pl.BlockSpec((H, D), lambda: (0, 0)),
                   pl.BlockSpec((8, 128), lambda: (0, 0))],
        out_shape=[jax.ShapeDtypeStruct((B, H), jnp.float32),
                   jax.ShapeDtypeStruct((H, D), jnp.float32),
                   jax.ShapeDtypeStruct((8, 128), jnp.float32)],
    )(stats, gath, intend.astype(jnp.float32),
      W_u.astype(jnp.float32), W_v.astype(jnp.float32),
      W_i.astype(jnp.float32),
      b_v.reshape(1, H).astype(jnp.float32),
      b_i.reshape(1, H).astype(jnp.float32),
      bn_gamma.reshape(1, D).astype(jnp.float32),
      bn_beta.reshape(1, D).astype(jnp.float32),
      W_e.reshape(1, H).astype(jnp.float32))

    ids3 = segment_ids.astype(jnp.int32).reshape(K4_STEPS, 1, K4_BLK)
    rst, _den = pl.pallas_call(
        _main_kernel,
        grid=(K4_STEPS,),
        in_specs=[pl.BlockSpec((K4_BLK, D), lambda i: (i, 0)),
                  pl.BlockSpec((1, 1, K4_BLK), lambda i: (i, 0, 0)),
                  pl.BlockSpec((B, H), lambda i: (0, 0)),
                  pl.BlockSpec((H, D), lambda i: (0, 0)),
                  pl.BlockSpec((8, 128), lambda i: (0, 0))],
        out_specs=[pl.BlockSpec((B, D), lambda i: (0, 0)),
                   pl.BlockSpec((B, 1), lambda i: (0, 0))],
        out_shape=[jax.ShapeDtypeStruct((B, D), jnp.float32),
                   jax.ShapeDtypeStruct((B, 1), jnp.float32)],
    )(feat, ids3, cvec, wu2, params)

    return rst


# speedup vs baseline: 5.1587x; 1.8129x over previous
import functools

import jax
import jax.numpy as jnp
from jax.experimental import pallas as pl
from jax.experimental.pallas import tpu as pltpu

N = 100000
B = 1024
D = 128
H = 128

K1_BLK = 10000
K4_BLK = 2000
K1_STEPS = N // K1_BLK
K4_STEPS = N // K4_BLK


def _stats_kernel(feat_ref, out_ref):
    i = pl.program_id(0)

    @pl.when(i == 0)
    def _():
        out_ref[...] = jnp.zeros_like(out_ref)

    f = feat_ref[...]
    s = jnp.sum(f, axis=0)
    q = jnp.sum(f * f, axis=0)
    upd = jnp.concatenate(
        [s[None, :], q[None, :], jnp.zeros((6, D), dtype=jnp.float32)], axis=0)
    out_ref[...] += upd


GATHER_G = 16


def _gather_kernel(last_ref, *refs):
    del last_ref
    ins, out_ref = refs[:GATHER_G], refs[GATHER_G]
    out_ref[...] = jnp.concatenate([r[...].reshape(1, D) for r in ins], axis=0)


def _prep_kernel(stats_ref, gath_ref, intend_ref, wu_ref, wv_ref, wi_ref,
                 bv_ref, bi_ref, gamma_ref, beta_ref, we_ref,
                 cvec_ref, wu2_ref, params_ref):
    stats = stats_ref[...]
    mean = stats[0:1, :] / N
    ex2 = stats[1:2, :] / N
    var = ex2 - mean * mean
    g = gamma_ref[...] * jax.lax.rsqrt(var + 1e-5)
    boff = beta_ref[...] - mean * g

    wu = wu_ref[...]
    wu2 = wu * g
    bu2 = jax.lax.dot_general(boff, wu, (((1,), (1,)), ((), ())),
                              preferred_element_type=jnp.float32)

    wi2 = wi_ref[...] * g
    bi2 = bi_ref[...] + jax.lax.dot_general(boff, wi_ref[...],
                                            (((1,), (1,)), ((), ())),
                                            preferred_element_type=jnp.float32)

    cvec = (jax.lax.dot_general(gath_ref[...], wi2, (((1,), (1,)), ((), ())),
                                preferred_element_type=jnp.float32)
            + jax.lax.dot_general(intend_ref[...], wv_ref[...],
                                  (((1,), (1,)), ((), ())),
                                  preferred_element_type=jnp.float32)
            + bv_ref[...] + bi2)

    cvec_ref[...] = cvec.astype(jnp.bfloat16)
    wu2_ref[...] = wu2.astype(jnp.bfloat16)
    params_ref[...] = jnp.concatenate(
        [bu2, g, boff, we_ref[...],
         jnp.zeros((4, 128), dtype=jnp.float32)], axis=0)


def _main_kernel(feat_ref, ids_ref, cvec_ref, wu2_ref, params_ref, we_ref,
                 rst_ref, den_ref):
    i = pl.program_id(0)

    @pl.when(i == 0)
    def _():
        rst_ref[...] = jnp.zeros_like(rst_ref)
        den_ref[...] = jnp.zeros_like(den_ref)

    f = feat_ref[...]
    u = jax.lax.dot_general(f.astype(jnp.bfloat16), wu2_ref[...],
                            (((1,), (1,)), ((), ())),
                            preferred_element_type=jnp.float32)
    u = u + params_ref[0:1, :]

    ids = ids_ref[0, 0, :]
    mask = (ids[:, None] ==
            jax.lax.broadcasted_iota(jnp.int32, (K4_BLK, B), 1))
    onehot = mask.astype(jnp.float32)
    cmat = jnp.dot(mask.astype(jnp.bfloat16), cvec_ref[...],
                   preferred_element_type=jnp.float32)

    s = jax.nn.sigmoid(u + cmat)
    e = jnp.dot(s, we_ref[...], preferred_element_type=jnp.float32)
    w = jnp.exp(e)

    rst_ref[...] += jax.lax.dot_general(onehot, f * w,
                                        (((0,), (0,)), ((), ())),
                                        preferred_element_type=jnp.float32)
    den_ref[...] += jax.lax.dot_general(onehot, w,
                                        (((0,), (0,)), ((), ())),
                                        preferred_element_type=jnp.float32)

    @pl.when(i == K4_STEPS - 1)
    def _():
        den = den_ref[...]
        g = params_ref[1:2, :]
        boff = params_ref[2:3, :]
        safe = jnp.where(den > 0, den, 1.0)
        rst_ref[...] = jnp.where(den > 0,
                                 rst_ref[...] / safe * g + boff,
                                 0.0)


@jax.jit
def kernel(feat, intend, last_nodes, segment_ids, bn_gamma, bn_beta,
           W_u, W_v, b_v, W_i, b_i, W_e):
    feat = feat.astype(jnp.float32)

    stats = pl.pallas_call(
        _stats_kernel,
        grid=(K1_STEPS,),
        in_specs=[pl.BlockSpec((K1_BLK, D), lambda i: (i, 0))],
        out_specs=pl.BlockSpec((8, D), lambda i: (0, 0)),
        out_shape=jax.ShapeDtypeStruct((8, D), jnp.float32),
    )(feat)

    feat3 = feat.reshape(N, 1, D)
    gath = pl.pallas_call(
        _gather_kernel,
        grid_spec=pltpu.PrefetchScalarGridSpec(
            num_scalar_prefetch=1,
            grid=(B // GATHER_G,),
            in_specs=[pl.BlockSpec(
                (1, 1, D),
                functools.partial(
                    lambda j, i, last_ref: (last_ref[i * GATHER_G + j], 0, 0),
                    j))
                for j in range(GATHER_G)],
            out_specs=pl.BlockSpec((GATHER_G, D),
                                   lambda i, last_ref: (i, 0)),
        ),
        out_shape=jax.ShapeDtypeStruct((B, D), jnp.float32),
    )(last_nodes.astype(jnp.int32), *([feat3] * GATHER_G))

    cvec, wu2, params = pl.pallas_call(
        _prep_kernel,
        in_specs=[pl.BlockSpec((8, D), lambda: (0, 0)),
                  pl.BlockSpec((B, D), lambda: (0, 0)),
                  pl.BlockSpec((B, D), lambda: (0, 0)),
                  pl.BlockSpec((H, D), lambda: (0, 0)),
                  pl.BlockSpec((H, D), lambda: (0, 0)),
                  pl.BlockSpec((H, D), lambda: (0, 0)),
                  pl.BlockSpec((1, H), lambda: (0, 0)),
                  pl.BlockSpec((1, H), lambda: (0, 0)),
                  pl.BlockSpec((1, D), lambda: (0, 0)),
                  pl.BlockSpec((1, D), lambda: (0, 0)),
                  pl.BlockSpec((1, H), lambda: (0, 0))],
        out_specs=[pl.BlockSpec((B, H), lambda: (0, 0)),
                   pl.BlockSpec((H, D), lambda: (0, 0)),
                   pl.BlockSpec((8, 128), lambda: (0, 0))],
        out_shape=[jax.ShapeDtypeStruct((B, H), jnp.bfloat16),
                   jax.ShapeDtypeStruct((H, D), jnp.bfloat16),
                   jax.ShapeDtypeStruct((8, 128), jnp.float32)],
    )(stats, gath, intend.astype(jnp.float32),
      W_u.astype(jnp.float32), W_v.astype(jnp.float32),
      W_i.astype(jnp.float32),
      b_v.reshape(1, H).astype(jnp.float32),
      b_i.reshape(1, H).astype(jnp.float32),
      bn_gamma.reshape(1, D).astype(jnp.float32),
      bn_beta.reshape(1, D).astype(jnp.float32),
      W_e.reshape(1, H).astype(jnp.float32))

    ids3 = segment_ids.astype(jnp.int32).reshape(K4_STEPS, 1, K4_BLK)
    rst, _den = pl.pallas_call(
        _main_kernel,
        grid=(K4_STEPS,),
        in_specs=[pl.BlockSpec((K4_BLK, D), lambda i: (i, 0)),
                  pl.BlockSpec((1, 1, K4_BLK), lambda i: (i, 0, 0)),
                  pl.BlockSpec((B, H), lambda i: (0, 0)),
                  pl.BlockSpec((H, D), lambda i: (0, 0)),
                  pl.BlockSpec((8, 128), lambda i: (0, 0)),
                  pl.BlockSpec((H, 1), lambda i: (0, 0))],
        out_specs=[pl.BlockSpec((B, D), lambda i: (0, 0)),
                   pl.BlockSpec((B, 1), lambda i: (0, 0))],
        out_shape=[jax.ShapeDtypeStruct((B, D), jnp.float32),
                   jax.ShapeDtypeStruct((B, 1), jnp.float32)],
    )(feat, ids3, cvec, wu2, params,
      W_e.reshape(H, 1).astype(jnp.float32))

    return rst


# speedup vs baseline: 5.4077x; 1.0483x over previous
import functools

import jax
import jax.numpy as jnp
from jax.experimental import pallas as pl
from jax.experimental.pallas import tpu as pltpu
from jax.experimental.pallas import tpu_sc as plsc

N = 100000
B = 1024
D = 128
H = 128

K1_BLK = 10000
K4_BLK = 2000
K1_STEPS = N // K1_BLK
K4_STEPS = N // K4_BLK


def _stats_kernel(feat_ref, out_ref):
    i = pl.program_id(0)

    @pl.when(i == 0)
    def _():
        out_ref[...] = jnp.zeros_like(out_ref)

    f = feat_ref[...]
    s = jnp.sum(f, axis=0)
    q = jnp.sum(f * f, axis=0)
    upd = jnp.concatenate(
        [s[None, :], q[None, :], jnp.zeros((6, D), dtype=jnp.float32)], axis=0)
    out_ref[...] += upd


_SC_INFO = plsc.get_sparse_core_info()
_NW = _SC_INFO.num_cores * _SC_INFO.num_subcores
_BPW = B // _NW

_sc_mesh = plsc.VectorSubcoreMesh(core_axis_name="c", subcore_axis_name="s")


@functools.partial(
    pl.kernel, mesh=_sc_mesh,
    out_type=jax.ShapeDtypeStruct((B, D), jnp.float32),
    scratch_types=[pltpu.VMEM((_BPW,), jnp.int32),
                   pltpu.VMEM((_BPW, D), jnp.float32),
                   pltpu.SemaphoreType.DMA],
)
def _sc_gather(table_hbm, idx_hbm, out_hbm, idx_v, rows_v, sem):
    wid = jax.lax.axis_index("s") * _SC_INFO.num_cores + jax.lax.axis_index("c")
    base = wid * _BPW
    pltpu.sync_copy(idx_hbm.at[pl.ds(base, _BPW)], idx_v)
    pltpu.async_copy(table_hbm.at[idx_v], rows_v, sem).wait()
    pltpu.sync_copy(rows_v, out_hbm.at[pl.ds(base, _BPW)])


def _prep_kernel(stats_ref, gath_ref, intend_ref, wu_ref, wv_ref, wi_ref,
                 bv_ref, bi_ref, gamma_ref, beta_ref, we_ref,
                 cvec_ref, wu2_ref, params_ref):
    stats = stats_ref[...]
    mean = stats[0:1, :] / N
    ex2 = stats[1:2, :] / N
    var = ex2 - mean * mean
    g = gamma_ref[...] * jax.lax.rsqrt(var + 1e-5)
    boff = beta_ref[...] - mean * g

    wu = wu_ref[...]
    wu2 = wu * g
    bu2 = jax.lax.dot_general(boff, wu, (((1,), (1,)), ((), ())),
                              preferred_element_type=jnp.float32)

    wi2 = wi_ref[...] * g
    bi2 = bi_ref[...] + jax.lax.dot_general(boff, wi_ref[...],
                                            (((1,), (1,)), ((), ())),
                                            preferred_element_type=jnp.float32)

    cvec = (jax.lax.dot_general(gath_ref[...], wi2, (((1,), (1,)), ((), ())),
                                preferred_element_type=jnp.float32)
            + jax.lax.dot_general(intend_ref[...], wv_ref[...],
                                  (((1,), (1,)), ((), ())),
                                  preferred_element_type=jnp.float32)
            + bv_ref[...] + bi2)

    cvec_ref[...] = cvec.astype(jnp.bfloat16)
    wu2_ref[...] = wu2.astype(jnp.bfloat16)
    params_ref[...] = jnp.concatenate(
        [bu2, g, boff, we_ref[...],
         jnp.zeros((4, 128), dtype=jnp.float32)], axis=0)


def _main_kernel(feat_ref, ids_ref, cvec_ref, wu2_ref, params_ref, we_ref,
                 rst_ref, den_ref):
    i = pl.program_id(0)

    @pl.when(i == 0)
    def _():
        rst_ref[...] = jnp.zeros_like(rst_ref)
        den_ref[...] = jnp.zeros_like(den_ref)

    f = feat_ref[...]
    u = jax.lax.dot_general(f.astype(jnp.bfloat16), wu2_ref[...],
                            (((1,), (1,)), ((), ())),
                            preferred_element_type=jnp.float32)
    u = u + params_ref[0:1, :]

    ids = ids_ref[0, 0, :]
    mask = (ids[:, None] ==
            jax.lax.broadcasted_iota(jnp.int32, (K4_BLK, B), 1))
    onehot = mask.astype(jnp.float32)
    cmat = jnp.dot(mask.astype(jnp.bfloat16), cvec_ref[...],
                   preferred_element_type=jnp.float32)

    s = jax.nn.sigmoid(u + cmat)
    e = jnp.dot(s, we_ref[...], preferred_element_type=jnp.float32)
    w = jnp.exp(e)

    rst_ref[...] += jax.lax.dot_general(onehot, f * w,
                                        (((0,), (0,)), ((), ())),
                                        preferred_element_type=jnp.float32)
    den_ref[...] += jax.lax.dot_general(onehot, w,
                                        (((0,), (0,)), ((), ())),
                                        preferred_element_type=jnp.float32)

    @pl.when(i == K4_STEPS - 1)
    def _():
        den = den_ref[...]
        g = params_ref[1:2, :]
        boff = params_ref[2:3, :]
        safe = jnp.where(den > 0, den, 1.0)
        rst_ref[...] = jnp.where(den > 0,
                                 rst_ref[...] / safe * g + boff,
                                 0.0)


@jax.jit
def kernel(feat, intend, last_nodes, segment_ids, bn_gamma, bn_beta,
           W_u, W_v, b_v, W_i, b_i, W_e):
    feat = feat.astype(jnp.float32)

    stats = pl.pallas_call(
        _stats_kernel,
        grid=(K1_STEPS,),
        in_specs=[pl.BlockSpec((K1_BLK, D), lambda i: (i, 0))],
        out_specs=pl.BlockSpec((8, D), lambda i: (0, 0)),
        out_shape=jax.ShapeDtypeStruct((8, D), jnp.float32),
    )(feat)

    gath = _sc_gather(feat, last_nodes.astype(jnp.int32))

    cvec, wu2, params = pl.pallas_call(
        _prep_kernel,
        in_specs=[pl.BlockSpec((8, D), lambda: (0, 0)),
                  pl.BlockSpec((B, D), lambda: (0, 0)),
                  pl.BlockSpec((B, D), lambda: (0, 0)),
                  pl.BlockSpec((H, D), lambda: (0, 0)),
                  pl.BlockSpec((H, D), lambda: (0, 0)),
                  pl.BlockSpec((H, D), lambda: (0, 0)),
                  pl.BlockSpec((1, H), lambda: (0, 0)),
                  pl.BlockSpec((1, H), lambda: (0, 0)),
                  pl.BlockSpec((1, D), lambda: (0, 0)),
                  pl.BlockSpec((1, D), lambda: (0, 0)),
                  pl.BlockSpec((1, H), lambda: (0, 0))],
        out_specs=[pl.BlockSpec((B, H), lambda: (0, 0)),
                   pl.BlockSpec((H, D), lambda: (0, 0)),
                   pl.BlockSpec((8, 128), lambda: (0, 0))],
        out_shape=[jax.ShapeDtypeStruct((B, H), jnp.bfloat16),
                   jax.ShapeDtypeStruct((H, D), jnp.bfloat16),
                   jax.ShapeDtypeStruct((8, 128), jnp.float32)],
    )(stats, gath, intend.astype(jnp.float32),
      W_u.astype(jnp.float32), W_v.astype(jnp.float32),
      W_i.astype(jnp.float32),
      b_v.reshape(1, H).astype(jnp.float32),
      b_i.reshape(1, H).astype(jnp.float32),
      bn_gamma.reshape(1, D).astype(jnp.float32),
      bn_beta.reshape(1, D).astype(jnp.float32),
      W_e.reshape(1, H).astype(jnp.float32))

    ids3 = segment_ids.astype(jnp.int32).reshape(K4_STEPS, 1, K4_BLK)
    rst, _den = pl.pallas_call(
        _main_kernel,
        grid=(K4_STEPS,),
        in_specs=[pl.BlockSpec((K4_BLK, D), lambda i: (i, 0)),
                  pl.BlockSpec((1, 1, K4_BLK), lambda i: (i, 0, 0)),
                  pl.BlockSpec((B, H), lambda i: (0, 0)),
                  pl.BlockSpec((H, D), lambda i: (0, 0)),
                  pl.BlockSpec((8, 128), lambda i: (0, 0)),
                  pl.BlockSpec((H, 1), lambda i: (0, 0))],
        out_specs=[pl.BlockSpec((B, D), lambda i: (0, 0)),
                   pl.BlockSpec((B, 1), lambda i: (0, 0))],
        out_shape=[jax.ShapeDtypeStruct((B, D), jnp.float32),
                   jax.ShapeDtypeStruct((B, 1), jnp.float32)],
    )(feat, ids3, cvec, wu2, params,
      W_e.reshape(H, 1).astype(jnp.float32))

    return rst


# speedup vs baseline: 6.2563x; 1.1569x over previous
import functools

import jax
import jax.numpy as jnp
from jax.experimental import pallas as pl
from jax.experimental.pallas import tpu as pltpu
from jax.experimental.pallas import tpu_sc as plsc

N = 100000
B = 1024
D = 128
H = 128

K1_BLK = 10000
K4_BLK = 4000
K1_STEPS = N // K1_BLK
K4_STEPS = N // K4_BLK


def _stats_kernel(feat_ref, out_ref):
    i = pl.program_id(0)

    @pl.when(i == 0)
    def _():
        out_ref[...] = jnp.zeros_like(out_ref)

    f = feat_ref[...]
    s = jnp.sum(f, axis=0)
    q = jnp.sum(f * f, axis=0)
    upd = jnp.concatenate(
        [s[None, :], q[None, :], jnp.zeros((6, D), dtype=jnp.float32)], axis=0)
    out_ref[...] += upd


_SC_INFO = plsc.get_sparse_core_info()
_NW = _SC_INFO.num_cores * _SC_INFO.num_subcores
_BPW = B // _NW

_sc_mesh = plsc.VectorSubcoreMesh(core_axis_name="c", subcore_axis_name="s")


@functools.partial(
    pl.kernel, mesh=_sc_mesh,
    out_type=jax.ShapeDtypeStruct((B, D), jnp.float32),
    scratch_types=[pltpu.VMEM((_BPW,), jnp.int32),
                   pltpu.VMEM((_BPW, D), jnp.float32),
                   pltpu.SemaphoreType.DMA],
)
def _sc_gather(table_hbm, idx_hbm, out_hbm, idx_v, rows_v, sem):
    wid = jax.lax.axis_index("s") * _SC_INFO.num_cores + jax.lax.axis_index("c")
    base = wid * _BPW
    pltpu.sync_copy(idx_hbm.at[pl.ds(base, _BPW)], idx_v)
    pltpu.async_copy(table_hbm.at[idx_v], rows_v, sem).wait()
    pltpu.sync_copy(rows_v, out_hbm.at[pl.ds(base, _BPW)])


def _prep_kernel(stats_ref, gath_ref, intend_ref, wu_ref, wv_ref, wi_ref,
                 bv_ref, bi_ref, gamma_ref, beta_ref, we_ref,
                 cvec_ref, wu2_ref, params_ref):
    stats = stats_ref[...]
    mean = stats[0:1, :] / N
    ex2 = stats[1:2, :] / N
    var = ex2 - mean * mean
    g = gamma_ref[...] * jax.lax.rsqrt(var + 1e-5)
    boff = beta_ref[...] - mean * g

    wu = wu_ref[...]
    wu2 = wu * g
    bu2 = jax.lax.dot_general(boff, wu, (((1,), (1,)), ((), ())),
                              preferred_element_type=jnp.float32)

    wi2 = wi_ref[...] * g
    bi2 = bi_ref[...] + jax.lax.dot_general(boff, wi_ref[...],
                                            (((1,), (1,)), ((), ())),
                                            preferred_element_type=jnp.float32)

    cvec = (jax.lax.dot_general(gath_ref[...], wi2, (((1,), (1,)), ((), ())),
                                preferred_element_type=jnp.float32)
            + jax.lax.dot_general(intend_ref[...], wv_ref[...],
                                  (((1,), (1,)), ((), ())),
                                  preferred_element_type=jnp.float32)
            + bv_ref[...] + bi2)

    cvec_ref[...] = cvec.astype(jnp.bfloat16)
    wu2_ref[...] = wu2.astype(jnp.bfloat16)
    params_ref[...] = jnp.concatenate(
        [bu2, g, boff, we_ref[...],
         jnp.zeros((4, 128), dtype=jnp.float32)], axis=0)


def _main_kernel(feat_ref, ids_ref, cvec_ref, wu2_ref, params_ref, we_ref,
                 rst_ref, den_ref):
    i = pl.program_id(0)

    @pl.when(i == 0)
    def _():
        rst_ref[...] = jnp.zeros_like(rst_ref)
        den_ref[...] = jnp.zeros_like(den_ref)

    f = feat_ref[...]
    u = jax.lax.dot_general(f.astype(jnp.bfloat16), wu2_ref[...],
                            (((1,), (1,)), ((), ())),
                            preferred_element_type=jnp.float32)
    u = u + params_ref[0:1, :]

    ids = ids_ref[0, 0, :]
    mask = (ids[:, None] ==
            jax.lax.broadcasted_iota(jnp.int32, (K4_BLK, B), 1))
    onehot = mask.astype(jnp.bfloat16)
    cmat = jnp.dot(onehot, cvec_ref[...],
                   preferred_element_type=jnp.float32)

    s = jax.nn.sigmoid(u + cmat)
    e = jnp.dot(s, we_ref[...], preferred_element_type=jnp.float32)
    w = jnp.exp(e)

    fw = (f * w).astype(jnp.bfloat16)
    rst_ref[...] += jax.lax.dot_general(onehot, fw,
                                        (((0,), (0,)), ((), ())),
                                        preferred_element_type=jnp.float32)
    den_ref[...] += jax.lax.dot_general(onehot, w.astype(jnp.bfloat16),
                                        (((0,), (0,)), ((), ())),
                                        preferred_element_type=jnp.float32)

    @pl.when(i == K4_STEPS - 1)
    def _():
        den = den_ref[...]
        g = params_ref[1:2, :]
        boff = params_ref[2:3, :]
        safe = jnp.where(den > 0, den, 1.0)
        rst_ref[...] = jnp.where(den > 0,
                                 rst_ref[...] / safe * g + boff,
                                 0.0)


@jax.jit
def kernel(feat, intend, last_nodes, segment_ids, bn_gamma, bn_beta,
           W_u, W_v, b_v, W_i, b_i, W_e):
    feat = feat.astype(jnp.float32)

    stats = pl.pallas_call(
        _stats_kernel,
        grid=(K1_STEPS,),
        in_specs=[pl.BlockSpec((K1_BLK, D), lambda i: (i, 0))],
        out_specs=pl.BlockSpec((8, D), lambda i: (0, 0)),
        out_shape=jax.ShapeDtypeStruct((8, D), jnp.float32),
    )(feat)

    gath = _sc_gather(feat, last_nodes.astype(jnp.int32))

    cvec, wu2, params = pl.pallas_call(
        _prep_kernel,
        in_specs=[pl.BlockSpec((8, D), lambda: (0, 0)),
                  pl.BlockSpec((B, D), lambda: (0, 0)),
                  pl.BlockSpec((B, D), lambda: (0, 0)),
                  pl.BlockSpec((H, D), lambda: (0, 0)),
                  pl.BlockSpec((H, D), lambda: (0, 0)),
                  pl.BlockSpec((H, D), lambda: (0, 0)),
                  pl.BlockSpec((1, H), lambda: (0, 0)),
                  pl.BlockSpec((1, H), lambda: (0, 0)),
                  pl.BlockSpec((1, D), lambda: (0, 0)),
                  pl.BlockSpec((1, D), lambda: (0, 0)),
                  pl.BlockSpec((1, H), lambda: (0, 0))],
        out_specs=[pl.BlockSpec((B, H), lambda: (0, 0)),
                   pl.BlockSpec((H, D), lambda: (0, 0)),
                   pl.BlockSpec((8, 128), lambda: (0, 0))],
        out_shape=[jax.ShapeDtypeStruct((B, H), jnp.bfloat16),
                   jax.ShapeDtypeStruct((H, D), jnp.bfloat16),
                   jax.ShapeDtypeStruct((8, 128), jnp.float32)],
    )(stats, gath, intend.astype(jnp.float32),
      W_u.astype(jnp.float32), W_v.astype(jnp.float32),
      W_i.astype(jnp.float32),
      b_v.reshape(1, H).astype(jnp.float32),
      b_i.reshape(1, H).astype(jnp.float32),
      bn_gamma.reshape(1, D).astype(jnp.float32),
      bn_beta.reshape(1, D).astype(jnp.float32),
      W_e.reshape(1, H).astype(jnp.float32))

    ids3 = segment_ids.astype(jnp.int32).reshape(K4_STEPS, 1, K4_BLK)
    rst, _den = pl.pallas_call(
        _main_kernel,
        grid=(K4_STEPS,),
        in_specs=[pl.BlockSpec((K4_BLK, D), lambda i: (i, 0)),
                  pl.BlockSpec((1, 1, K4_BLK), lambda i: (i, 0, 0)),
                  pl.BlockSpec((B, H), lambda i: (0, 0)),
                  pl.BlockSpec((H, D), lambda i: (0, 0)),
                  pl.BlockSpec((8, 128), lambda i: (0, 0)),
                  pl.BlockSpec((H, 1), lambda i: (0, 0))],
        out_specs=[pl.BlockSpec((B, D), lambda i: (0, 0)),
                   pl.BlockSpec((B, 1), lambda i: (0, 0))],
        out_shape=[jax.ShapeDtypeStruct((B, D), jnp.float32),
                   jax.ShapeDtypeStruct((B, 1), jnp.float32)],
    )(feat, ids3, cvec, wu2, params,
      W_e.reshape(H, 1).astype(jnp.float32))

    return rst


# speedup vs baseline: 6.9431x; 1.1098x over previous
import functools

import jax
import jax.numpy as jnp
from jax.experimental import pallas as pl
from jax.experimental.pallas import tpu as pltpu
from jax.experimental.pallas import tpu_sc as plsc

N = 100000
B = 1024
D = 128
H = 128

K1_BLK = 10000
K4_BLK = 4000
K1_STEPS = N // K1_BLK
K4_STEPS = N // K4_BLK


def _stats_kernel(feat_ref, out_ref):
    i = pl.program_id(0)

    @pl.when(i == 0)
    def _():
        out_ref[...] = jnp.zeros_like(out_ref)

    f = feat_ref[...]
    s = jnp.sum(f, axis=0)
    q = jnp.sum(f * f, axis=0)
    upd = jnp.concatenate(
        [s[None, :], q[None, :], jnp.zeros((6, D), dtype=jnp.float32)], axis=0)
    out_ref[...] += upd


_SC_INFO = plsc.get_sparse_core_info()
_NW = _SC_INFO.num_cores * _SC_INFO.num_subcores
_BPW = B // _NW

_sc_mesh = plsc.VectorSubcoreMesh(core_axis_name="c", subcore_axis_name="s")


@functools.partial(
    pl.kernel, mesh=_sc_mesh,
    out_type=jax.ShapeDtypeStruct((B, D), jnp.float32),
    scratch_types=[pltpu.VMEM((_BPW,), jnp.int32),
                   pltpu.VMEM((_BPW, D), jnp.float32),
                   pltpu.SemaphoreType.DMA],
)
def _sc_gather(table_hbm, idx_hbm, out_hbm, idx_v, rows_v, sem):
    wid = jax.lax.axis_index("s") * _SC_INFO.num_cores + jax.lax.axis_index("c")
    base = wid * _BPW
    pltpu.sync_copy(idx_hbm.at[pl.ds(base, _BPW)], idx_v)
    pltpu.async_copy(table_hbm.at[idx_v], rows_v, sem).wait()
    pltpu.sync_copy(rows_v, out_hbm.at[pl.ds(base, _BPW)])


def _prep_kernel(stats_ref, gath_ref, intend_ref, wu_ref, wv_ref, wi_ref,
                 bv_ref, bi_ref, gamma_ref, beta_ref, we_ref,
                 cvec_ref, wu2_ref, params_ref):
    stats = stats_ref[...]
    mean = stats[0:1, :] / N
    ex2 = stats[1:2, :] / N
    var = ex2 - mean * mean
    g = gamma_ref[...] * jax.lax.rsqrt(var + 1e-5)
    boff = beta_ref[...] - mean * g

    wu = wu_ref[...]
    wu2 = wu * g
    bu2 = jax.lax.dot_general(boff, wu, (((1,), (1,)), ((), ())),
                              preferred_element_type=jnp.float32)

    wi2 = wi_ref[...] * g
    bi2 = bi_ref[...] + jax.lax.dot_general(boff, wi_ref[...],
                                            (((1,), (1,)), ((), ())),
                                            preferred_element_type=jnp.float32)

    cvec = (jax.lax.dot_general(gath_ref[...], wi2, (((1,), (1,)), ((), ())),
                                preferred_element_type=jnp.float32)
            + jax.lax.dot_general(intend_ref[...], wv_ref[...],
                                  (((1,), (1,)), ((), ())),
                                  preferred_element_type=jnp.float32)
            + bv_ref[...] + bi2)

    cvec_ref[...] = cvec.astype(jnp.bfloat16)
    wu2_ref[...] = wu2.astype(jnp.bfloat16)
    params_ref[...] = jnp.concatenate(
        [bu2, g, boff, we_ref[...],
         jnp.zeros((4, 128), dtype=jnp.float32)], axis=0)


def _main_kernel(feat_ref, ids_ref, cvec_ref, wu2_ref, params_ref, we_ref,
                 acc_ref):
    i = pl.program_id(0)

    @pl.when(i == 0)
    def _():
        acc_ref[...] = jnp.zeros_like(acc_ref)

    f = feat_ref[...]
    u = jax.lax.dot_general(f.astype(jnp.bfloat16), wu2_ref[...],
                            (((1,), (1,)), ((), ())),
                            preferred_element_type=jnp.float32)
    u = u + params_ref[0:1, :]

    ids = ids_ref[0, 0, :]
    mask = (ids[:, None] ==
            jax.lax.broadcasted_iota(jnp.int32, (K4_BLK, B), 1))
    onehot = mask.astype(jnp.bfloat16)
    cmat = jnp.dot(onehot, cvec_ref[...],
                   preferred_element_type=jnp.float32)

    s = jax.nn.sigmoid(u + cmat)
    e = jnp.dot(s, we_ref[...], preferred_element_type=jnp.float32)
    w = jnp.exp(e)

    aug = jnp.concatenate(
        [f * w, w, jnp.zeros((K4_BLK, 127), jnp.float32)],
        axis=1).astype(jnp.bfloat16)
    acc_ref[...] += jax.lax.dot_general(onehot, aug,
                                        (((0,), (0,)), ((), ())),
                                        preferred_element_type=jnp.float32)

    @pl.when(i == K4_STEPS - 1)
    def _():
        acc = acc_ref[...]
        den = acc[:, D:D + 1]
        g = params_ref[1:2, :]
        boff = params_ref[2:3, :]
        safe = jnp.where(den > 0, den, 1.0)
        final = jnp.where(den > 0,
                          acc[:, 0:D] / safe * g + boff,
                          0.0)
        acc_ref[...] = jnp.concatenate([final, acc[:, D:]], axis=1)


@jax.jit
def kernel(feat, intend, last_nodes, segment_ids, bn_gamma, bn_beta,
           W_u, W_v, b_v, W_i, b_i, W_e):
    feat = feat.astype(jnp.float32)

    stats = pl.pallas_call(
        _stats_kernel,
        grid=(K1_STEPS,),
        in_specs=[pl.BlockSpec((K1_BLK, D), lambda i: (i, 0))],
        out_specs=pl.BlockSpec((8, D), lambda i: (0, 0)),
        out_shape=jax.ShapeDtypeStruct((8, D), jnp.float32),
    )(feat)

    gath = _sc_gather(feat, last_nodes.astype(jnp.int32))

    cvec, wu2, params = pl.pallas_call(
        _prep_kernel,
        in_specs=[pl.BlockSpec((8, D), lambda: (0, 0)),
                  pl.BlockSpec((B, D), lambda: (0, 0)),
                  pl.BlockSpec((B, D), lambda: (0, 0)),
                  pl.BlockSpec((H, D), lambda: (0, 0)),
                  pl.BlockSpec((H, D), lambda: (0, 0)),
                  pl.BlockSpec((H, D), lambda: (0, 0)),
                  pl.BlockSpec((1, H), lambda: (0, 0)),
                  pl.BlockSpec((1, H), lambda: (0, 0)),
                  pl.BlockSpec((1, D), lambda: (0, 0)),
                  pl.BlockSpec((1, D), lambda: (0, 0)),
                  pl.BlockSpec((1, H), lambda: (0, 0))],
        out_specs=[pl.BlockSpec((B, H), lambda: (0, 0)),
                   pl.BlockSpec((H, D), lambda: (0, 0)),
                   pl.BlockSpec((8, 128), lambda: (0, 0))],
        out_shape=[jax.ShapeDtypeStruct((B, H), jnp.bfloat16),
                   jax.ShapeDtypeStruct((H, D), jnp.bfloat16),
                   jax.ShapeDtypeStruct((8, 128), jnp.float32)],
    )(stats, gath, intend.astype(jnp.float32),
      W_u.astype(jnp.float32), W_v.astype(jnp.float32),
      W_i.astype(jnp.float32),
      b_v.reshape(1, H).astype(jnp.float32),
      b_i.reshape(1, H).astype(jnp.float32),
      bn_gamma.reshape(1, D).astype(jnp.float32),
      bn_beta.reshape(1, D).astype(jnp.float32),
      W_e.reshape(1, H).astype(jnp.float32))

    ids3 = segment_ids.astype(jnp.int32).reshape(K4_STEPS, 1, K4_BLK)
    acc = pl.pallas_call(
        _main_kernel,
        grid=(K4_STEPS,),
        in_specs=[pl.BlockSpec((K4_BLK, D), lambda i: (i, 0)),
                  pl.BlockSpec((1, 1, K4_BLK), lambda i: (i, 0, 0)),
                  pl.BlockSpec((B, H), lambda i: (0, 0)),
                  pl.BlockSpec((H, D), lambda i: (0, 0)),
                  pl.BlockSpec((8, 128), lambda i: (0, 0)),
                  pl.BlockSpec((H, 1), lambda i: (0, 0))],
        out_specs=pl.BlockSpec((B, 256), lambda i: (0, 0)),
        out_shape=jax.ShapeDtypeStruct((B, 256), jnp.float32),
    )(feat, ids3, cvec, wu2, params,
      W_e.reshape(H, 1).astype(jnp.float32))

    return acc[:, 0:D]
